# baseline (speedup 1.0000x reference)
import jax
import jax.numpy as jnp
from jax import lax
from jax.experimental import pallas as pl
from jax.experimental.pallas import tpu as pltpu
from jax.experimental.pallas import tpu_sc as plsc

_EXPERTS = 16
_THRESH = 1.0
_LEAK = 0.9
_DT = 0.1
_RC = 10

_MM_BM = 512
_SC_CHUNK = 1024
_UNROLL = 8
_SM_BM = 512


def _matmul_body(x_ref, w_ref, out_ref):
    out_ref[...] = jnp.dot(x_ref[...], w_ref[...],
                           preferred_element_type=jnp.float32) * _DT


def _matmul(x, w):
    n, h = x.shape
    e = w.shape[1]
    return pl.pallas_call(
        _matmul_body,
        grid=(n // _MM_BM,),
        in_specs=[
            pl.BlockSpec((_MM_BM, h), lambda i: (i, 0)),
            pl.BlockSpec((h, e), lambda i: (0, 0)),
        ],
        out_specs=pl.BlockSpec((_MM_BM, e), lambda i: (i, 0)),
        out_shape=jax.ShapeDtypeStruct((n, e), jnp.float32),
    )(x, w)


def _rot(v, k, rot_idx):
    return v.at[rot_idx[k]].get(mode="promise_in_bounds")


def _scan_body(cdt_hbm, rout_hbm, cur_a, cur_b, out_v, stage, memsl, rout_v,
               sem_a, sem_b):
    core = lax.axis_index("c")
    sub = lax.axis_index("s")
    n = cdt_hbm.shape[0] // _EXPERTS
    nch = n // _SC_CHUNK
    cw = _SC_CHUNK * _EXPERTS
    wid = core * 16 + sub

    @pl.when(wid == 0)
    def _():
        bufs = (cur_a, cur_b)
        sems = (sem_a, sem_b)
        cps = [None] * nch
        for c in range(min(2, nch)):
            cps[c] = pltpu.async_copy(
                cdt_hbm.at[pl.ds(c * cw, cw)], bufs[c].at[pl.ds(0, cw)],
                sems[c])

        mp = jnp.zeros((_EXPERTS,), jnp.float32)
        rc = jnp.zeros((_EXPERTS,), jnp.int32)
        for c in range(nch):
            buf = bufs[c % 2]
            cps[c].wait()
            base = c * _SC_CHUNK
            cg = jnp.where(rc <= 1, buf[pl.ds(0, _EXPERTS)], 0.0)

            def body(i, carry, buf=buf, base=base):
                mp, rc, cg = carry
                for j in range(_UNROLL):
                    t = i * _UNROLL + j
                    spike = mp > _THRESH
                    mp = jnp.where(spike, 0.0, mp * _LEAK + cg)
                    out_v[pl.ds((base + t) * _EXPERTS, _EXPERTS)] = mp
                    rc = jnp.where(spike, _RC, rc - 1)
                    cg = jnp.where(rc <= 1,
                                   buf[pl.ds((t + 1) * _EXPERTS, _EXPERTS)],
                                   0.0)
                return (mp, rc, cg)

            mp, rc, cg = lax.fori_loop(0, _SC_CHUNK // _UNROLL, body,
                                       (mp, rc, cg))
            if c + 2 < nch:
                cps[c + 2] = pltpu.async_copy(
                    cdt_hbm.at[pl.ds((c + 2) * cw, cw)],
                    buf.at[pl.ds(0, cw)], sems[c % 2])

        pltpu.sync_copy(out_v, stage)

    @pl.when(core == 0)
    def _():
        plsc.subcore_barrier()
        tw = n // 16 * _EXPERTS
        pltpu.sync_copy(stage.at[pl.ds(sub * tw, tw)], memsl)
        lanes = lax.iota(jnp.int32, 16)
        rot_idx = {k: (lanes + k) & 15 for k in (1, 2, 4, 8)}

        def tok(i, _):
            for j in range(4):
                t = i * 4 + j
                m = memsl[pl.ds(t * _EXPERTS, _EXPERTS)]
                spike = m > _THRESH
                sw = jnp.where(spike, 1.0, 0.0)
                mx = m
                for k in (8, 4, 2, 1):
                    mx = jnp.maximum(mx, _rot(mx, k, rot_idx))
                anyv = mx > _THRESH
                ex = jnp.exp(jnp.where(anyv, sw, m))
                s = ex
                for k in (8, 4, 2, 1):
                    s = s + _rot(s, k, rot_idx)
                rout_v[pl.ds(t * _EXPERTS, _EXPERTS)] = ex / s
            return 0

        lax.fori_loop(0, n // 16 // 4, tok, 0)
        pltpu.sync_copy(rout_v, rout_hbm.at[sub])


def _scan_softmax(cdt):
    n, e = cdt.shape
    mesh = plsc.VectorSubcoreMesh(core_axis_name="c", subcore_axis_name="s")
    flat = cdt.reshape(n * e)
    return pl.kernel(
        _scan_body,
        out_type=jax.ShapeDtypeStruct((16, n * e // 16), jnp.float32),
        mesh=mesh,
        scratch_types=[
            pltpu.VMEM((_SC_CHUNK * _EXPERTS + _EXPERTS,), jnp.float32),
            pltpu.VMEM((_SC_CHUNK * _EXPERTS + _EXPERTS,), jnp.float32),
            pltpu.VMEM((n * e,), jnp.float32),
            pltpu.VMEM_SHARED((n * e,), jnp.float32),
            pltpu.VMEM((n * e // 16,), jnp.float32),
            pltpu.VMEM((n * e // 16,), jnp.float32),
            pltpu.SemaphoreType.DMA,
            pltpu.SemaphoreType.DMA,
        ],
    )(flat)


def _routing_body(mem_ref, out_ref):
    mem = mem_ref[...]
    spike = mem > _THRESH
    any_spike = jnp.any(spike, axis=-1, keepdims=True)
    sel = jnp.where(any_spike, spike.astype(jnp.float32), mem / _THRESH)
    out_ref[...] = jax.nn.softmax(sel, axis=-1)


def _routing(mem_pre):
    n, e = mem_pre.shape
    return pl.pallas_call(
        _routing_body,
        grid=(n // _SM_BM,),
        in_specs=[pl.BlockSpec((_SM_BM, e), lambda i: (i, 0))],
        out_specs=pl.BlockSpec((_SM_BM, e), lambda i: (i, 0)),
        out_shape=jax.ShapeDtypeStruct((n, e), jnp.float32),
    )(mem_pre)


@jax.jit
def kernel(hidden_states, synaptic_weights):
    b, s, h = hidden_states.shape
    e = synaptic_weights.shape[1]
    x = hidden_states.reshape(b * s, h)
    cdt = _matmul(x, synaptic_weights)
    rout_flat = _scan_softmax(cdt)
    return rout_flat.reshape(b, s, e)


# speedup vs baseline: 1.0459x; 1.0459x over previous
import jax
import jax.numpy as jnp
from jax import lax
from jax.experimental import pallas as pl
from jax.experimental.pallas import tpu as pltpu
from jax.experimental.pallas import tpu_sc as plsc

_EXPERTS = 16
_THRESH = 1.0
_LEAK = 0.9
_DT = 0.1
_RC = 10

_MM_BM = 512
_SC_CHUNK = 1024
_UNROLL = 8
_SM_BM = 512


def _matmul_body(x_ref, wt_ref, out_ref):
    out_ref[...] = lax.dot_general(
        x_ref[...], wt_ref[...], (((1,), (1,)), ((), ())),
        preferred_element_type=jnp.float32) * _DT


def _matmul(x, wt):
    n, h = x.shape
    e = wt.shape[0]
    return pl.pallas_call(
        _matmul_body,
        grid=(n // _MM_BM,),
        in_specs=[
            pl.BlockSpec((_MM_BM, h), lambda i: (i, 0)),
            pl.BlockSpec((e, h), lambda i: (0, 0)),
        ],
        out_specs=pl.BlockSpec((_MM_BM, e), lambda i: (i, 0)),
        out_shape=jax.ShapeDtypeStruct((n, e), jnp.float32),
    )(x, wt)


def _rot(v, k, rot_idx):
    return v.at[rot_idx[k]].get(mode="promise_in_bounds")


def _scan_body(cdt_hbm, rout_hbm, cur_a, cur_b, out_v, stage, memsl, rout_v,
               sem_a, sem_b):
    core = lax.axis_index("c")
    sub = lax.axis_index("s")
    n = cdt_hbm.shape[0] // _EXPERTS
    nch = n // _SC_CHUNK
    cw = _SC_CHUNK * _EXPERTS
    wid = core * 16 + sub

    @pl.when(wid == 0)
    def _():
        bufs = (cur_a, cur_b)
        sems = (sem_a, sem_b)
        cps = [None] * nch
        for c in range(min(2, nch)):
            cps[c] = pltpu.async_copy(
                cdt_hbm.at[pl.ds(c * cw, cw)], bufs[c].at[pl.ds(0, cw)],
                sems[c])

        mp = jnp.zeros((_EXPERTS,), jnp.float32)
        rc = jnp.zeros((_EXPERTS,), jnp.int32)
        for c in range(nch):
            buf = bufs[c % 2]
            cps[c].wait()
            base = c * _SC_CHUNK
            cg = jnp.where(rc <= 1, buf[pl.ds(0, _EXPERTS)], 0.0)

            def body(i, carry, buf=buf, base=base):
                mp, rc, cg = carry
                for j in range(_UNROLL):
                    t = i * _UNROLL + j
                    spike = mp > _THRESH
                    mp = jnp.where(spike, 0.0, mp * _LEAK + cg)
                    out_v[pl.ds((base + t) * _EXPERTS, _EXPERTS)] = mp
                    rc = jnp.where(spike, _RC, rc - 1)
                    cg = jnp.where(rc <= 1,
                                   buf[pl.ds((t + 1) * _EXPERTS, _EXPERTS)],
                                   0.0)
                return (mp, rc, cg)

            mp, rc, cg = lax.fori_loop(0, _SC_CHUNK // _UNROLL, body,
                                       (mp, rc, cg))
            if c + 2 < nch:
                cps[c + 2] = pltpu.async_copy(
                    cdt_hbm.at[pl.ds((c + 2) * cw, cw)],
                    buf.at[pl.ds(0, cw)], sems[c % 2])

        pltpu.sync_copy(out_v, stage)

    @pl.when(core == 0)
    def _():
        plsc.subcore_barrier()
        tw = n // 16 * _EXPERTS
        pltpu.sync_copy(stage.at[pl.ds(sub * tw, tw)], memsl)
        lanes = lax.iota(jnp.int32, 16)
        rot_idx = {k: (lanes + k) & 15 for k in (1, 2, 4, 8)}

        def tok(i, _):
            for j in range(4):
                t = i * 4 + j
                m = memsl[pl.ds(t * _EXPERTS, _EXPERTS)]
                spike = m > _THRESH
                sw = jnp.where(spike, 1.0, 0.0)
                mx = m
                for k in (8, 4, 2, 1):
                    mx = jnp.maximum(mx, _rot(mx, k, rot_idx))
                anyv = mx > _THRESH
                ex = jnp.exp(jnp.where(anyv, sw, m))
                s = ex
                for k in (8, 4, 2, 1):
                    s = s + _rot(s, k, rot_idx)
                rout_v[pl.ds(t * _EXPERTS, _EXPERTS)] = ex / s
            return 0

        lax.fori_loop(0, n // 16 // 4, tok, 0)
        pltpu.sync_copy(rout_v, rout_hbm.at[sub])


def _scan_softmax(cdt):
    n, e = cdt.shape
    mesh = plsc.VectorSubcoreMesh(core_axis_name="c", subcore_axis_name="s")
    flat = cdt.reshape(n * e)
    return pl.kernel(
        _scan_body,
        out_type=jax.ShapeDtypeStruct((16, n * e // 16), jnp.float32),
        mesh=mesh,
        scratch_types=[
            pltpu.VMEM((_SC_CHUNK * _EXPERTS + _EXPERTS,), jnp.float32),
            pltpu.VMEM((_SC_CHUNK * _EXPERTS + _EXPERTS,), jnp.float32),
            pltpu.VMEM((n * e,), jnp.float32),
            pltpu.VMEM_SHARED((n * e,), jnp.float32),
            pltpu.VMEM((n * e // 16,), jnp.float32),
            pltpu.VMEM((n * e // 16,), jnp.float32),
            pltpu.SemaphoreType.DMA,
            pltpu.SemaphoreType.DMA,
        ],
    )(flat)


def _routing_body(mem_ref, out_ref):
    mem = mem_ref[...]
    spike = mem > _THRESH
    any_spike = jnp.any(spike, axis=-1, keepdims=True)
    sel = jnp.where(any_spike, spike.astype(jnp.float32), mem / _THRESH)
    out_ref[...] = jax.nn.softmax(sel, axis=-1)


def _routing(mem_pre):
    n, e = mem_pre.shape
    return pl.pallas_call(
        _routing_body,
        grid=(n // _SM_BM,),
        in_specs=[pl.BlockSpec((_SM_BM, e), lambda i: (i, 0))],
        out_specs=pl.BlockSpec((_SM_BM, e), lambda i: (i, 0)),
        out_shape=jax.ShapeDtypeStruct((n, e), jnp.float32),
    )(mem_pre)


@jax.jit
def kernel(hidden_states, synaptic_weights):
    b, s, h = hidden_states.shape
    e = synaptic_weights.shape[1]
    x = hidden_states.reshape(b * s, h)
    cdt = _matmul(x, synaptic_weights.T)
    rout_flat = _scan_softmax(cdt)
    return rout_flat.reshape(b, s, e)
